# baseline (speedup 1.0000x reference)
import functools

import jax
import jax.numpy as jnp
from jax import lax
from jax.experimental import pallas as pl
from jax.experimental.pallas import tpu as pltpu
from jax.experimental.pallas import tpu_sc as plsc

N_FIELDS = 26
VOCAB = 100000
DIM = 16
BATCH = 16384

NC = 2
NS = 16
NW = NC * NS
B_PER_W = BATCH // NW
CHUNK = 128
NCH = B_PER_W // CHUNK
FC = N_FIELDS * CHUNK
L = 16


@functools.partial(
    pl.kernel,
    mesh=plsc.VectorSubcoreMesh(core_axis_name="c", subcore_axis_name="s"),
    compiler_params=pltpu.CompilerParams(
        needs_layout_passes=False, use_tc_tiling_on_sc=False),
    out_type=jax.ShapeDtypeStruct((BATCH,), jnp.float32),
    scratch_types=[
        pltpu.VMEM((FC // 128, 128), jnp.int32),
        pltpu.VMEM((FC, DIM), jnp.float32),
        pltpu.VMEM((FC,), jnp.float32),
        pltpu.VMEM((CHUNK,), jnp.float32),
        pltpu.SemaphoreType.DMA,
        pltpu.SemaphoreType.DMA,
    ],
)
def _fm_sc(x_hbm, emb_hbm, lin_hbm, out_hbm,
           idx_v, emb_v, lin_v, outb_v, sem_e, sem_l):
    wid = lax.axis_index("s") * NC + lax.axis_index("c")
    lane = lax.iota(jnp.int32, L)
    lane_f = lane * N_FIELDS

    NR = FC // 128

    for c in range(NCH):
        base = wid * B_PER_W + c * CHUNK
        pltpu.sync_copy(x_hbm.at[pl.ds(wid * (NCH * NR) + c * NR, NR)], idx_v)

        def fire(r, _):
            pltpu.async_copy(emb_hbm.at[idx_v.at[r]],
                             emb_v.at[pl.ds(r * 128, 128)], sem_e)
            pltpu.async_copy(lin_hbm.at[idx_v.at[r]],
                             lin_v.at[pl.ds(r * 128, 128)], sem_l)
            return 0

        lax.fori_loop(0, NR, fire, 0)

        def drain(r, _):
            pltpu.make_async_copy(emb_hbm.at[idx_v.at[r]],
                                  emb_v.at[pl.ds(r * 128, 128)], sem_e).wait()
            pltpu.make_async_copy(lin_hbm.at[idx_v.at[r]],
                                  lin_v.at[pl.ds(r * 128, 128)], sem_l).wait()
            return 0

        lax.fori_loop(0, NR, drain, 0)

        def group_body(g, _):
            s0f = g * (L * N_FIELDS)
            zero = jnp.zeros((L,), jnp.float32)

            def dim_body(d, carry):
                cross, ssq = carry
                dcol = jnp.full((L,), d, jnp.int32)
                sd = zero
                for f in range(N_FIELDS):
                    rows = lane_f + (s0f + f)
                    v = plsc.load_gather(emb_v, [rows, dcol])
                    sd = sd + v
                    ssq = ssq + v * v
                return cross + sd * sd, ssq

            cross, ssq = lax.fori_loop(0, DIM, dim_body, (zero, zero))
            res = (cross - ssq) * 0.5
            for f in range(N_FIELDS):
                rows = lane_f + (s0f + f)
                res = res + plsc.load_gather(lin_v, [rows])
            outb_v[pl.ds(g * L, L)] = 1.0 / (1.0 + jnp.exp(-res))
            return 0

        lax.fori_loop(0, CHUNK // L, group_body, 0)
        plsc.subcore_barrier()
        pltpu.sync_copy(outb_v, out_hbm.at[pl.ds(base, CHUNK)])


def kernel(x, emb_table, lin_table):
    offsets = (jnp.arange(N_FIELDS) * VOCAB).astype(x.dtype)
    flat = (x + offsets[None, :]).astype(jnp.int32).reshape(-1, 128)
    out = _fm_sc(flat, emb_table, lin_table.reshape(-1))
    return out.reshape(BATCH, 1)
